# baseline (speedup 1.0000x reference)
import functools

import jax
import jax.numpy as jnp
from jax import lax
from jax.experimental import pallas as pl
from jax.experimental.pallas import tpu as pltpu
from jax.experimental.pallas import tpu_sc as plsc

_NUM_EMB = 1024
_D = 256
_CC = 0.25
_BLK = 2048

_NC = 2
_NS = 16
_NW = _NC * _NS


def _vq_body(flat_ref, w_ref, idx_ref, loss_ref, perp_ref,
             wsq_ref, counts_ref, sumsq_ref):
    step = pl.program_id(0)
    nsteps = pl.num_programs(0)
    flat = flat_ref[...]
    w = w_ref[...]

    @pl.when(step == 0)
    def _init():
        wsq_ref[...] = jnp.sum(w * w, axis=1).reshape(1, _NUM_EMB)
        counts_ref[...] = jnp.zeros_like(counts_ref)
        sumsq_ref[0] = 0.0

    flat_sq = jnp.sum(flat * flat, axis=1, keepdims=True)
    mm2 = jax.lax.dot_general(-2.0 * flat, w, (((1,), (1,)), ((), ())),
                              preferred_element_type=jnp.float32)
    dist = (flat_sq + wsq_ref[...]) + mm2

    minval = jnp.min(dist, axis=1, keepdims=True)
    iota = jax.lax.broadcasted_iota(jnp.int32, (_BLK, _NUM_EMB), 1)
    idx = jnp.min(jnp.where(dist == minval, iota, _NUM_EMB), axis=1)

    idx_ref[...] = idx.reshape(idx_ref.shape)

    onehot = (iota == idx[:, None]).astype(jnp.bfloat16)
    counts_ref[...] += jnp.dot(jnp.ones((8, _BLK), jnp.bfloat16), onehot,
                               preferred_element_type=jnp.float32)
    sumsq_ref[0] += jnp.sum(minval)

    @pl.when(step == nsteps - 1)
    def _fin():
        n_rows = nsteps * _BLK
        m = sumsq_ref[0] / (n_rows * _D)
        loss_ref[...] = jnp.full((1, 1), m + _CC * m, jnp.float32)
        avg = counts_ref[0:1, :] / n_rows
        perp = jnp.exp(-jnp.sum(avg * jnp.log(avg + 1e-10)))
        perp_ref[...] = jnp.full((1, 1), perp, jnp.float32)


def _tc_part(flat, W):
    n = flat.shape[0]
    grid = n // _BLK
    return pl.pallas_call(
        _vq_body,
        grid=(grid,),
        in_specs=[
            pl.BlockSpec((_BLK, _D), lambda i: (i, 0)),
            pl.BlockSpec((_NUM_EMB, _D), lambda i: (0, 0)),
        ],
        out_specs=[
            pl.BlockSpec((_BLK // 128, 128), lambda i: (i, 0)),
            pl.BlockSpec((1, 1), lambda i: (0, 0)),
            pl.BlockSpec((1, 1), lambda i: (0, 0)),
        ],
        out_shape=[
            jax.ShapeDtypeStruct((n // 128, 128), jnp.int32),
            jax.ShapeDtypeStruct((1, 1), jnp.float32),
            jax.ShapeDtypeStruct((1, 1), jnp.float32),
        ],
        scratch_shapes=[
            pltpu.VMEM((1, _NUM_EMB), jnp.float32),
            pltpu.VMEM((8, _NUM_EMB), jnp.float32),
            pltpu.SMEM((1,), jnp.float32),
        ],
    )(flat, W)


def _sc_gather(table, idx):
    b = idx.shape[0]
    bpw = b // _NW
    ch = 160
    nch = bpw // ch
    mesh = plsc.VectorSubcoreMesh(core_axis_name="c", subcore_axis_name="s",
                                  num_cores=_NC, num_subcores=_NS)

    @functools.partial(
        pl.kernel, mesh=mesh,
        out_type=jax.ShapeDtypeStruct((b, _D), jnp.float32),
        scratch_types=[
            pltpu.VMEM((bpw,), jnp.int32),
            pltpu.VMEM((ch, _D), jnp.float32),
            pltpu.VMEM((ch, _D), jnp.float32),
            pltpu.SemaphoreType.DMA,
            pltpu.SemaphoreType.DMA,
        ],
    )
    def k(table_hbm, idx_hbm, out_hbm, idx_v, rows_a, rows_b, sem_a, sem_b):
        wid = lax.axis_index("s") * _NC + lax.axis_index("c")
        base = wid * bpw
        pltpu.sync_copy(idx_hbm.at[pl.ds(base, bpw)], idx_v)
        bufs = ((rows_a, sem_a), (rows_b, sem_b))
        pending = pltpu.async_copy(
            table_hbm.at[idx_v.at[pl.ds(0, ch)]], rows_a, sem_a)
        for c in range(nch):
            rows, _ = bufs[c % 2]
            nxt = None
            if c + 1 < nch:
                nrows, nsem = bufs[(c + 1) % 2]
                nxt = pltpu.async_copy(
                    table_hbm.at[idx_v.at[pl.ds((c + 1) * ch, ch)]],
                    nrows, nsem)
            pending.wait()
            pltpu.sync_copy(rows, out_hbm.at[pl.ds(base + c * ch, ch)])
            pending = nxt

    return k(table, idx)


def kernel(inputs, W):
    input_shape = inputs.shape
    flat = inputs.reshape(-1, _D)

    idx2d, loss, perp = _tc_part(flat, W)
    q = _sc_gather(W, idx2d.reshape(-1))

    quantized = q.reshape(input_shape[0], -1)
    encoding_indices = idx2d.reshape(input_shape[0], -1)
    return (loss[0, 0], quantized, perp[0, 0], encoding_indices)


# speedup vs baseline: 1.1356x; 1.1356x over previous
import jax
import jax.numpy as jnp
from jax.experimental import pallas as pl
from jax.experimental.pallas import tpu as pltpu

_NUM_EMB = 1024
_D = 256
_CC = 0.25
_BLK = 2048


def _vq_body(flat_ref, w_ref, q_ref, idx_ref, loss_ref, perp_ref,
             wsq_ref, counts_ref, sumsq_ref):
    step = pl.program_id(0)
    nsteps = pl.num_programs(0)
    flat = flat_ref[...]
    w = w_ref[...]

    @pl.when(step == 0)
    def _init():
        wsq_ref[...] = jnp.sum(w * w, axis=1).reshape(1, _NUM_EMB)
        counts_ref[...] = jnp.zeros_like(counts_ref)
        sumsq_ref[0] = 0.0

    flat_sq = jnp.sum(flat * flat, axis=1, keepdims=True)
    mm2 = jax.lax.dot_general(-2.0 * flat, w, (((1,), (1,)), ((), ())),
                              preferred_element_type=jnp.float32)
    dist = (flat_sq + wsq_ref[...]) + mm2

    minval = jnp.min(dist, axis=1, keepdims=True)
    iotaf = jax.lax.broadcasted_iota(
        jnp.int32, (1, _NUM_EMB), 1).astype(jnp.float32)
    idxf = jnp.min(jnp.where(dist == minval, iotaf, float(_NUM_EMB)), axis=1)
    idx = idxf.astype(jnp.int32)

    onehot = (iotaf == idxf[:, None]).astype(jnp.bfloat16)
    q = jnp.dot(onehot, w.astype(jnp.bfloat16),
                preferred_element_type=jnp.float32)

    q_ref[...] = q
    idx_ref[...] = idx.reshape(idx_ref.shape)

    counts_ref[...] += jnp.dot(jnp.ones((8, _BLK), jnp.bfloat16), onehot,
                               preferred_element_type=jnp.float32)
    sumsq_ref[0] += jnp.sum(minval)

    @pl.when(step == nsteps - 1)
    def _fin():
        n_rows = nsteps * _BLK
        m = sumsq_ref[0] / (n_rows * _D)
        loss_ref[...] = jnp.full((1, 1), m + _CC * m, jnp.float32)
        avg = counts_ref[0:1, :] / n_rows
        perp = jnp.exp(-jnp.sum(avg * jnp.log(avg + 1e-10)))
        perp_ref[...] = jnp.full((1, 1), perp, jnp.float32)


def kernel(inputs, W):
    input_shape = inputs.shape
    flat = inputs.reshape(-1, _D)
    n = flat.shape[0]
    grid = n // _BLK

    q, idx2d, loss, perp = pl.pallas_call(
        _vq_body,
        grid=(grid,),
        in_specs=[
            pl.BlockSpec((_BLK, _D), lambda i: (i, 0)),
            pl.BlockSpec((_NUM_EMB, _D), lambda i: (0, 0)),
        ],
        out_specs=[
            pl.BlockSpec((_BLK, _D), lambda i: (i, 0)),
            pl.BlockSpec((_BLK // 128, 128), lambda i: (i, 0)),
            pl.BlockSpec((1, 1), lambda i: (0, 0)),
            pl.BlockSpec((1, 1), lambda i: (0, 0)),
        ],
        out_shape=[
            jax.ShapeDtypeStruct((n, _D), jnp.float32),
            jax.ShapeDtypeStruct((n // 128, 128), jnp.int32),
            jax.ShapeDtypeStruct((1, 1), jnp.float32),
            jax.ShapeDtypeStruct((1, 1), jnp.float32),
        ],
        scratch_shapes=[
            pltpu.VMEM((1, _NUM_EMB), jnp.float32),
            pltpu.VMEM((8, _NUM_EMB), jnp.float32),
            pltpu.SMEM((1,), jnp.float32),
        ],
    )(flat, W)

    quantized = q.reshape(input_shape[0], -1)
    encoding_indices = idx2d.reshape(input_shape[0], -1)
    return (loss[0, 0], quantized, perp[0, 0], encoding_indices)


# speedup vs baseline: 1.1590x; 1.0206x over previous
import jax
import jax.numpy as jnp
from jax.experimental import pallas as pl
from jax.experimental.pallas import tpu as pltpu

_NUM_EMB = 1024
_D = 256
_CC = 0.25
_BLK = 4096


def _vq_body(flat_ref, w_ref, q_ref, idx_ref, loss_ref, perp_ref,
             wsq_ref, counts_ref, sumsq_ref):
    step = pl.program_id(0)
    nsteps = pl.num_programs(0)
    flat = flat_ref[...]
    w = w_ref[...]

    @pl.when(step == 0)
    def _init():
        wsq_ref[...] = jnp.sum(w * w, axis=1).reshape(1, _NUM_EMB)
        counts_ref[...] = jnp.zeros_like(counts_ref)
        sumsq_ref[0] = 0.0

    flat_sq = jnp.sum(flat * flat, axis=1, keepdims=True)
    mm2 = jax.lax.dot_general(-2.0 * flat, w, (((1,), (1,)), ((), ())),
                              preferred_element_type=jnp.float32)
    dist = (flat_sq + wsq_ref[...]) + mm2

    minval = jnp.min(dist, axis=1, keepdims=True)
    iotaf = jax.lax.broadcasted_iota(
        jnp.int32, (1, _NUM_EMB), 1).astype(jnp.float32)
    idxf = jnp.min(jnp.where(dist == minval, iotaf, float(_NUM_EMB)), axis=1)
    idx = idxf.astype(jnp.int32)

    onehot = (iotaf == idxf[:, None]).astype(jnp.bfloat16)
    q = jnp.dot(onehot, w.astype(jnp.bfloat16),
                preferred_element_type=jnp.float32)

    q_ref[...] = q
    idx_ref[...] = idx.reshape(idx_ref.shape)

    counts_ref[...] += jnp.dot(jnp.ones((8, _BLK), jnp.bfloat16), onehot,
                               preferred_element_type=jnp.float32)
    sumsq_ref[0] += jnp.sum(minval)

    @pl.when(step == nsteps - 1)
    def _fin():
        n_rows = nsteps * _BLK
        m = sumsq_ref[0] / (n_rows * _D)
        loss_ref[...] = jnp.full((1, 1), m + _CC * m, jnp.float32)
        avg = counts_ref[0:1, :] / n_rows
        perp = jnp.exp(-jnp.sum(avg * jnp.log(avg + 1e-10)))
        perp_ref[...] = jnp.full((1, 1), perp, jnp.float32)


def kernel(inputs, W):
    input_shape = inputs.shape
    flat = inputs.reshape(-1, _D)
    n = flat.shape[0]
    grid = n // _BLK

    q, idx2d, loss, perp = pl.pallas_call(
        _vq_body,
        grid=(grid,),
        in_specs=[
            pl.BlockSpec((_BLK, _D), lambda i: (i, 0)),
            pl.BlockSpec((_NUM_EMB, _D), lambda i: (0, 0)),
        ],
        out_specs=[
            pl.BlockSpec((_BLK, _D), lambda i: (i, 0)),
            pl.BlockSpec((_BLK // 128, 128), lambda i: (i, 0)),
            pl.BlockSpec((1, 1), lambda i: (0, 0)),
            pl.BlockSpec((1, 1), lambda i: (0, 0)),
        ],
        out_shape=[
            jax.ShapeDtypeStruct((n, _D), jnp.float32),
            jax.ShapeDtypeStruct((n // 128, 128), jnp.int32),
            jax.ShapeDtypeStruct((1, 1), jnp.float32),
            jax.ShapeDtypeStruct((1, 1), jnp.float32),
        ],
        scratch_shapes=[
            pltpu.VMEM((1, _NUM_EMB), jnp.float32),
            pltpu.VMEM((8, _NUM_EMB), jnp.float32),
            pltpu.SMEM((1,), jnp.float32),
        ],
    )(flat, W)

    quantized = q.reshape(input_shape[0], -1)
    encoding_indices = idx2d.reshape(input_shape[0], -1)
    return (loss[0, 0], quantized, perp[0, 0], encoding_indices)


# speedup vs baseline: 1.1650x; 1.0051x over previous
import jax
import jax.numpy as jnp
from jax.experimental import pallas as pl
from jax.experimental.pallas import tpu as pltpu

_NUM_EMB = 1024
_D = 256
_CC = 0.25
_BLK = 4096


def _vq_body(flat_ref, w_ref, q_ref, idx_ref, loss_ref, perp_ref,
             wsq_ref, wneg_ref, wbf_ref, counts_ref, sumsq_ref):
    step = pl.program_id(0)
    nsteps = pl.num_programs(0)
    flat = flat_ref[...]

    @pl.when(step == 0)
    def _init():
        w = w_ref[...]
        wsq_ref[...] = jnp.sum(w * w, axis=1).reshape(1, _NUM_EMB)
        wneg_ref[...] = -2.0 * w
        wbf_ref[...] = w.astype(jnp.bfloat16)
        counts_ref[...] = jnp.zeros_like(counts_ref)
        sumsq_ref[0] = 0.0

    mm2 = jax.lax.dot_general(flat, wneg_ref[...], (((1,), (1,)), ((), ())),
                              preferred_element_type=jnp.float32)
    flat_sq = jnp.sum(flat * flat, axis=1, keepdims=True)
    dist = (flat_sq + wsq_ref[...]) + mm2

    minval = jnp.min(dist, axis=1, keepdims=True)
    iotaf = jax.lax.broadcasted_iota(
        jnp.int32, (1, _NUM_EMB), 1).astype(jnp.float32)
    idxf = jnp.min(jnp.where(dist == minval, iotaf, float(_NUM_EMB)), axis=1)
    idx = idxf.astype(jnp.int32)

    onehot = (iotaf == idxf[:, None]).astype(jnp.bfloat16)
    q = jnp.dot(onehot, wbf_ref[...],
                preferred_element_type=jnp.float32)

    q_ref[...] = q
    idx_ref[...] = idx.reshape(idx_ref.shape)

    counts_ref[...] += jnp.dot(jnp.ones((8, _BLK), jnp.bfloat16), onehot,
                               preferred_element_type=jnp.float32)
    sumsq_ref[0] += jnp.sum(minval)

    @pl.when(step == nsteps - 1)
    def _fin():
        n_rows = nsteps * _BLK
        m = sumsq_ref[0] / (n_rows * _D)
        loss_ref[...] = jnp.full((1, 1), m + _CC * m, jnp.float32)
        avg = counts_ref[0:1, :] / n_rows
        perp = jnp.exp(-jnp.sum(avg * jnp.log(avg + 1e-10)))
        perp_ref[...] = jnp.full((1, 1), perp, jnp.float32)


def kernel(inputs, W):
    input_shape = inputs.shape
    flat = inputs.reshape(-1, _D)
    n = flat.shape[0]
    grid = n // _BLK

    q, idx2d, loss, perp = pl.pallas_call(
        _vq_body,
        grid=(grid,),
        in_specs=[
            pl.BlockSpec((_BLK, _D), lambda i: (i, 0)),
            pl.BlockSpec((_NUM_EMB, _D), lambda i: (0, 0)),
        ],
        out_specs=[
            pl.BlockSpec((_BLK, _D), lambda i: (i, 0)),
            pl.BlockSpec((_BLK // 128, 128), lambda i: (i, 0)),
            pl.BlockSpec((1, 1), lambda i: (0, 0)),
            pl.BlockSpec((1, 1), lambda i: (0, 0)),
        ],
        out_shape=[
            jax.ShapeDtypeStruct((n, _D), jnp.float32),
            jax.ShapeDtypeStruct((n // 128, 128), jnp.int32),
            jax.ShapeDtypeStruct((1, 1), jnp.float32),
            jax.ShapeDtypeStruct((1, 1), jnp.float32),
        ],
        scratch_shapes=[
            pltpu.VMEM((1, _NUM_EMB), jnp.float32),
            pltpu.VMEM((_NUM_EMB, _D), jnp.float32),
            pltpu.VMEM((_NUM_EMB, _D), jnp.bfloat16),
            pltpu.VMEM((8, _NUM_EMB), jnp.float32),
            pltpu.SMEM((1,), jnp.float32),
        ],
    )(flat, W)

    quantized = q.reshape(input_shape[0], -1)
    encoding_indices = idx2d.reshape(input_shape[0], -1)
    return (loss[0, 0], quantized, perp[0, 0], encoding_indices)


# speedup vs baseline: 1.1787x; 1.0118x over previous
import jax
import jax.numpy as jnp
from jax.experimental import pallas as pl
from jax.experimental.pallas import tpu as pltpu

_NUM_EMB = 1024
_D = 256
_CC = 0.25
_BLK = 4096


def _vq_body(flat_ref, w_ref, q_ref, idx_ref, loss_ref, perp_ref,
             wsq_ref, wneg_ref, counts_ref, sumsq_ref):
    step = pl.program_id(0)
    nsteps = pl.num_programs(0)
    flat = flat_ref[...]

    @pl.when(step == 0)
    def _init():
        w = w_ref[...]
        wsq_ref[...] = jnp.sum(w * w, axis=1).reshape(1, _NUM_EMB)
        wneg_ref[...] = -2.0 * w
        counts_ref[...] = jnp.zeros_like(counts_ref)
        sumsq_ref[0] = 0.0

    mm2 = jax.lax.dot_general(flat, wneg_ref[...], (((1,), (1,)), ((), ())),
                              preferred_element_type=jnp.float32)
    flat_sq = jnp.sum(flat * flat, axis=1, keepdims=True)
    dist = (flat_sq + wsq_ref[...]) + mm2

    minval = jnp.min(dist, axis=1, keepdims=True)
    iotaf = jax.lax.broadcasted_iota(
        jnp.int32, (1, _NUM_EMB), 1).astype(jnp.float32)
    idxf = jnp.min(jnp.where(dist == minval, iotaf, float(_NUM_EMB)), axis=1)
    idx = idxf.astype(jnp.int32)

    onehot = (iotaf == idxf[:, None]).astype(jnp.float32)
    q = jnp.dot(onehot, w_ref[...],
                preferred_element_type=jnp.float32)

    q_ref[...] = q
    idx_ref[...] = idx.reshape(idx_ref.shape)

    counts_ref[...] += jnp.dot(jnp.ones((8, _BLK), jnp.float32), onehot,
                               preferred_element_type=jnp.float32)
    sumsq_ref[0] += jnp.sum(minval)

    @pl.when(step == nsteps - 1)
    def _fin():
        n_rows = nsteps * _BLK
        m = sumsq_ref[0] / (n_rows * _D)
        loss_ref[...] = jnp.full((1, 1), m + _CC * m, jnp.float32)
        avg = counts_ref[0:1, :] / n_rows
        perp = jnp.exp(-jnp.sum(avg * jnp.log(avg + 1e-10)))
        perp_ref[...] = jnp.full((1, 1), perp, jnp.float32)


def kernel(inputs, W):
    input_shape = inputs.shape
    flat = inputs.reshape(-1, _D)
    n = flat.shape[0]
    grid = n // _BLK

    q, idx2d, loss, perp = pl.pallas_call(
        _vq_body,
        grid=(grid,),
        in_specs=[
            pl.BlockSpec((_BLK, _D), lambda i: (i, 0)),
            pl.BlockSpec((_NUM_EMB, _D), lambda i: (0, 0)),
        ],
        out_specs=[
            pl.BlockSpec((_BLK, _D), lambda i: (i, 0)),
            pl.BlockSpec((_BLK // 128, 128), lambda i: (i, 0)),
            pl.BlockSpec((1, 1), lambda i: (0, 0)),
            pl.BlockSpec((1, 1), lambda i: (0, 0)),
        ],
        out_shape=[
            jax.ShapeDtypeStruct((n, _D), jnp.float32),
            jax.ShapeDtypeStruct((n // 128, 128), jnp.int32),
            jax.ShapeDtypeStruct((1, 1), jnp.float32),
            jax.ShapeDtypeStruct((1, 1), jnp.float32),
        ],
        scratch_shapes=[
            pltpu.VMEM((1, _NUM_EMB), jnp.float32),
            pltpu.VMEM((_NUM_EMB, _D), jnp.float32),
            pltpu.VMEM((8, _NUM_EMB), jnp.float32),
            pltpu.SMEM((1,), jnp.float32),
        ],
    )(flat, W)

    quantized = q.reshape(input_shape[0], -1)
    encoding_indices = idx2d.reshape(input_shape[0], -1)
    return (loss[0, 0], quantized, perp[0, 0], encoding_indices)
